# baseline (speedup 1.0000x reference)
import functools

import jax
import jax.numpy as jnp
from jax import lax
from jax.experimental import pallas as pl
from jax.experimental.pallas import tpu as pltpu
from jax.experimental.pallas import tpu_sc as plsc

N = 10000
DIM = 128
NC = 2
NS = 16
NW = NC * NS
L = 16
CHUNK = 128
NBUF = 3
RPT = 624
TAIL = N - NS * RPT


def _sweep_body(z, colr, rowr, ewr, zrows, out,
                colv, rowv, ewv, rows, acc, *sems, nchunk):
    gs = sems[:NBUF]
    ss = sems[NBUF:2 * NBUF]
    isem = sems[2 * NBUF:]
    c = lax.axis_index("c")
    s = lax.axis_index("s")
    wid = c * NS + s
    tbase = wid * nchunk * CHUNK

    def idx_start(i, b):
        base = pl.multiple_of(tbase + i * CHUNK, 8)
        pltpu.async_copy(colr.at[pl.ds(base, CHUNK)], colv.at[b], isem[b])
        pltpu.async_copy(rowr.at[pl.ds(base, CHUNK)], rowv.at[b], isem[b])
        pltpu.async_copy(ewr.at[pl.ds(base, CHUNK)], ewv.at[b], isem[b])

    def idx_wait(b):
        pltpu.make_async_copy(colr.at[pl.ds(0, CHUNK)], colv.at[b],
                              isem[b]).wait()
        pltpu.make_async_copy(rowr.at[pl.ds(0, CHUNK)], rowv.at[b],
                              isem[b]).wait()
        pltpu.make_async_copy(ewr.at[pl.ds(0, CHUNK)], ewv.at[b],
                              isem[b]).wait()

    def gather_start(b):
        pltpu.async_copy(z.at[colv.at[b]], rows.at[b], gs[b])

    def gather_wait(b):
        pltpu.make_async_copy(z.at[colv.at[b]], rows.at[b], gs[b]).wait()

    def scatter_start(b):
        pltpu.async_copy(rows.at[b], acc.at[rowv.at[b]], ss[b], add=True)

    def scatter_wait(b):
        pltpu.make_async_copy(rows.at[b], acc.at[rowv.at[b]], ss[b]).wait()

    for b in range(NBUF - 1):
        idx_start(b, b)

    zbase = pl.multiple_of(s * RPT, 8)
    pltpu.sync_copy(zrows.at[pl.ds(0, RPT)], acc.at[pl.ds(zbase, RPT)])

    @pl.when(s == NS - 1)
    def _():
        pltpu.sync_copy(zrows.at[pl.ds(0, TAIL)], acc.at[pl.ds(NS * RPT, TAIL)])

    idx_wait(0)
    gather_start(0)
    plsc.subcore_barrier()

    def outer(g, carry):
        for b in range(NBUF):
            i = g * NBUF + b
            gather_wait(b)

            def grp(g2, carry2, _b=b):
                wvec = ewv[_b, pl.ds(g2 * L, L)]
                for t in range(L):
                    wv = jnp.full((L,), wvec[t], jnp.float32)
                    e = g2 * L + t
                    for j in range(DIM // L):
                        rows[_b, e, pl.ds(j * L, L)] = (
                            rows[_b, e, pl.ds(j * L, L)] * wv)
                return carry2

            lax.fori_loop(0, CHUNK // L, grp, 0)

            scatter_start(b)

            b1 = (b + 1) % NBUF

            @pl.when(i + 1 < nchunk)
            def _(b1=b1):
                idx_wait(b1)
                gather_start(b1)

            b3 = (b + NBUF - 1) % NBUF
            j3 = i + NBUF - 1

            @pl.when(j3 < nchunk)
            def _(b3=b3, j3=j3, i=i):
                @pl.when(i >= 1)
                def _():
                    scatter_wait(b3)

                idx_start(j3, b3)

        return carry

    lax.fori_loop(0, nchunk // NBUF, outer, 0)

    for b in range(NBUF):
        scatter_wait(b)

    plsc.subcore_barrier()
    dbase = pl.multiple_of(s * RPT, 8)
    pltpu.sync_copy(acc.at[pl.ds(dbase, RPT)], out.at[c, pl.ds(dbase, RPT)])

    @pl.when(s == NS - 1)
    def _():
        pltpu.sync_copy(acc.at[pl.ds(NS * RPT, TAIL)],
                        out.at[c, pl.ds(NS * RPT, TAIL)])


def _make_sweep(nchunk):
    mesh = plsc.VectorSubcoreMesh(core_axis_name="c", subcore_axis_name="s",
                                  num_cores=NC, num_subcores=NS)
    return pl.kernel(
        functools.partial(_sweep_body, nchunk=nchunk),
        out_type=jax.ShapeDtypeStruct((NC, N, DIM), jnp.float32),
        mesh=mesh,
        scratch_types=[
            pltpu.VMEM((NBUF, CHUNK), jnp.int32),
            pltpu.VMEM((NBUF, CHUNK), jnp.int32),
            pltpu.VMEM((NBUF, CHUNK), jnp.float32),
            pltpu.VMEM((NBUF, CHUNK, DIM), jnp.float32),
            pltpu.VMEM_SHARED((N, DIM), jnp.float32),
        ] + [pltpu.SemaphoreType.DMA] * (3 * NBUF),
    )


def _pre_body(x_ref, z_ref):
    x = x_ref[...]
    head = x[:, 0:1]
    tail = x[:, 1:]
    xk = tail / head
    n2 = jnp.clip(jnp.sum(xk * xk, axis=1, keepdims=True), 0.0, 0.9)
    lamb = 1.0 / jnp.sqrt(1.0 - n2)
    z_ref[...] = jnp.concatenate([lamb, lamb * xk], axis=1)


def _combine(p):
    a = p[0] + p[1]
    a0 = a[:, 0:1]
    inv = jnp.where(a0 != 0.0, 1.0 / a0, 0.0)
    km = a[:, 1:] * inv
    n2 = jnp.clip(jnp.sum(km * km, axis=1, keepdims=True), 0.0, 0.9)
    lamb = 1.0 / jnp.sqrt(1.0 - n2)
    pm = km * (lamb / (lamb + 1.0))
    alpha = 1.6732632423543772
    scale = 1.0507009873554805
    sp = scale * jnp.where(pm > 0, pm, alpha * (jnp.exp(pm) - 1.0))
    n2s = jnp.sum(sp * sp, axis=1, keepdims=True)
    denom = jnp.maximum(1.0 - n2s, 1e-6)
    xr = 2.0 * sp / denom
    headn = jnp.sqrt(1.0 + jnp.sum(xr * xr, axis=1, keepdims=True))
    return xr, headn


def _mid_body(p_ref, z_ref):
    xr, headn = _combine(p_ref[...])
    xk = xr / headn
    n2 = jnp.clip(jnp.sum(xk * xk, axis=1, keepdims=True), 0.0, 0.9)
    lamb = 1.0 / jnp.sqrt(1.0 - n2)
    z_ref[...] = jnp.concatenate([lamb, lamb * xk], axis=1)


def _post_body(p_ref, o_ref):
    xr, headn = _combine(p_ref[...])
    o_ref[...] = jnp.concatenate([headn, xr], axis=1)


_BLK = 1000


def _dense_pre(x):
    return pl.pallas_call(
        _pre_body,
        grid=(N // _BLK,),
        in_specs=[pl.BlockSpec((_BLK, DIM), lambda i: (i, 0))],
        out_specs=pl.BlockSpec((_BLK, DIM), lambda i: (i, 0)),
        out_shape=jax.ShapeDtypeStruct((N, DIM), jnp.float32),
    )(x)


def _dense_stage(body, p):
    return pl.pallas_call(
        body,
        grid=(N // _BLK,),
        in_specs=[pl.BlockSpec((NC, _BLK, DIM), lambda i: (0, i, 0))],
        out_specs=pl.BlockSpec((_BLK, DIM), lambda i: (i, 0)),
        out_shape=jax.ShapeDtypeStruct((N, DIM), jnp.float32),
    )(p)


def kernel(x, edge_index, edge_weight, msg_weight):
    del msg_weight
    row = edge_index[0]
    col = edge_index[1]
    e = edge_weight.shape[0]
    nchunk = -(-e // (NW * CHUNK))
    nchunk = -(-nchunk // NBUF) * NBUF
    pad = NW * nchunk * CHUNK - e
    if pad:
        row = jnp.pad(row, (0, pad))
        col = jnp.pad(col, (0, pad))
        edge_weight = jnp.pad(edge_weight, (0, pad))
    zrows = jnp.zeros((RPT, DIM), jnp.float32)

    sweep = _make_sweep(nchunk)
    z = _dense_pre(x)
    p = sweep(z, col, row, edge_weight, zrows)
    z = _dense_stage(_mid_body, p)
    p = sweep(z, col, row, edge_weight, zrows)
    return _dense_stage(_post_body, p)


# speedup vs baseline: 1.2445x; 1.2445x over previous
import functools

import jax
import jax.numpy as jnp
from jax import lax
from jax.experimental import pallas as pl
from jax.experimental.pallas import tpu as pltpu
from jax.experimental.pallas import tpu_sc as plsc

N = 10000
DIM = 128
NC = 2
NS = 16
NW = NC * NS
L = 16
CHUNK = 128
SB = 8
RPT = 624
TAIL = N - NS * RPT


def _sweep_body(z, colr, rowr, ewr, zrows, out,
                colb, rowb, ewb, rows, acc, gsem, isem0, isem1, *, nchunk):
    isem = (isem0, isem1)
    c = lax.axis_index("c")
    s = lax.axis_index("s")
    wid = c * NS + s
    nsb = nchunk // SB

    def idx_start(sb, p):
        base = pl.multiple_of(wid * nchunk + sb * SB, 8)
        pltpu.async_copy(colr.at[pl.ds(base, SB)], colb.at[p], isem[p])
        pltpu.async_copy(rowr.at[pl.ds(base, SB)], rowb.at[p], isem[p])
        pltpu.async_copy(ewr.at[pl.ds(base, SB)], ewb.at[p], isem[p])

    def idx_wait(p):
        pltpu.make_async_copy(colr.at[pl.ds(0, SB)], colb.at[p],
                              isem[p]).wait()
        pltpu.make_async_copy(rowr.at[pl.ds(0, SB)], rowb.at[p],
                              isem[p]).wait()
        pltpu.make_async_copy(ewr.at[pl.ds(0, SB)], ewb.at[p],
                              isem[p]).wait()

    idx_start(0, 0)

    zbase = pl.multiple_of(s * RPT, 8)
    pltpu.sync_copy(zrows.at[pl.ds(0, RPT)], acc.at[pl.ds(zbase, RPT)])

    @pl.when(s == NS - 1)
    def _():
        pltpu.sync_copy(zrows.at[pl.ds(0, TAIL)], acc.at[pl.ds(NS * RPT, TAIL)])

    plsc.subcore_barrier()

    def pair(gp, carry):
        for p in range(2):
            sb = gp * 2 + p
            idx_wait(p)

            @pl.when(sb + 1 < nsb)
            def _(sb=sb, p=p):
                idx_start(sb + 1, 1 - p)

            def chunk(j, carry2, _p=p):
                pltpu.async_copy(z.at[colb.at[_p, j]], rows, gsem).wait()

                def grp(g2, carry3):
                    wvec = ewb[_p, j, pl.ds(g2 * L, L)]
                    for t in range(L):
                        wv = jnp.full((L,), wvec[t], jnp.float32)
                        e = g2 * L + t
                        for k in range(DIM // L):
                            rows[e, pl.ds(k * L, L)] = (
                                rows[e, pl.ds(k * L, L)] * wv)
                    return carry3

                lax.fori_loop(0, CHUNK // L, grp, 0)

                pltpu.sync_copy(rows, acc.at[rowb.at[_p, j]], add=True)
                return carry2

            lax.fori_loop(0, SB, chunk, 0)
        return carry

    lax.fori_loop(0, nsb // 2, pair, 0)

    plsc.subcore_barrier()
    dbase = pl.multiple_of(s * RPT, 8)
    pltpu.sync_copy(acc.at[pl.ds(dbase, RPT)], out.at[c, pl.ds(dbase, RPT)])

    @pl.when(s == NS - 1)
    def _():
        pltpu.sync_copy(acc.at[pl.ds(NS * RPT, TAIL)],
                        out.at[c, pl.ds(NS * RPT, TAIL)])


def _make_sweep(nchunk):
    mesh = plsc.VectorSubcoreMesh(core_axis_name="c", subcore_axis_name="s",
                                  num_cores=NC, num_subcores=NS)
    return pl.kernel(
        functools.partial(_sweep_body, nchunk=nchunk),
        out_type=jax.ShapeDtypeStruct((NC, N, DIM), jnp.float32),
        mesh=mesh,
        scratch_types=[
            pltpu.VMEM((2, SB, CHUNK), jnp.int32),
            pltpu.VMEM((2, SB, CHUNK), jnp.int32),
            pltpu.VMEM((2, SB, CHUNK), jnp.float32),
            pltpu.VMEM((CHUNK, DIM), jnp.float32),
            pltpu.VMEM_SHARED((N, DIM), jnp.float32),
        ] + [pltpu.SemaphoreType.DMA] * 3,
    )


def _pre_body(x_ref, z_ref):
    x = x_ref[...]
    head = x[:, 0:1]
    tail = x[:, 1:]
    xk = tail / head
    n2 = jnp.clip(jnp.sum(xk * xk, axis=1, keepdims=True), 0.0, 0.9)
    lamb = 1.0 / jnp.sqrt(1.0 - n2)
    z_ref[...] = jnp.concatenate([lamb, lamb * xk], axis=1)


def _combine(p):
    a = p[0] + p[1]
    a0 = a[:, 0:1]
    inv = jnp.where(a0 != 0.0, 1.0 / a0, 0.0)
    km = a[:, 1:] * inv
    n2 = jnp.clip(jnp.sum(km * km, axis=1, keepdims=True), 0.0, 0.9)
    lamb = 1.0 / jnp.sqrt(1.0 - n2)
    pm = km * (lamb / (lamb + 1.0))
    alpha = 1.6732632423543772
    scale = 1.0507009873554805
    sp = scale * jnp.where(pm > 0, pm, alpha * (jnp.exp(pm) - 1.0))
    n2s = jnp.sum(sp * sp, axis=1, keepdims=True)
    denom = jnp.maximum(1.0 - n2s, 1e-6)
    xr = 2.0 * sp / denom
    headn = jnp.sqrt(1.0 + jnp.sum(xr * xr, axis=1, keepdims=True))
    return xr, headn


def _mid_body(p_ref, z_ref):
    xr, headn = _combine(p_ref[...])
    xk = xr / headn
    n2 = jnp.clip(jnp.sum(xk * xk, axis=1, keepdims=True), 0.0, 0.9)
    lamb = 1.0 / jnp.sqrt(1.0 - n2)
    z_ref[...] = jnp.concatenate([lamb, lamb * xk], axis=1)


def _post_body(p_ref, o_ref):
    xr, headn = _combine(p_ref[...])
    o_ref[...] = jnp.concatenate([headn, xr], axis=1)


_BLK = 1000


def _dense_pre(x):
    return pl.pallas_call(
        _pre_body,
        grid=(N // _BLK,),
        in_specs=[pl.BlockSpec((_BLK, DIM), lambda i: (i, 0))],
        out_specs=pl.BlockSpec((_BLK, DIM), lambda i: (i, 0)),
        out_shape=jax.ShapeDtypeStruct((N, DIM), jnp.float32),
    )(x)


def _dense_stage(body, p):
    return pl.pallas_call(
        body,
        grid=(N // _BLK,),
        in_specs=[pl.BlockSpec((NC, _BLK, DIM), lambda i: (0, i, 0))],
        out_specs=pl.BlockSpec((_BLK, DIM), lambda i: (i, 0)),
        out_shape=jax.ShapeDtypeStruct((N, DIM), jnp.float32),
    )(p)


def kernel(x, edge_index, edge_weight, msg_weight):
    del msg_weight
    row = edge_index[0]
    col = edge_index[1]
    e = edge_weight.shape[0]
    nchunk = -(-e // (NW * CHUNK))
    nchunk = -(-nchunk // (2 * SB)) * (2 * SB)
    pad = NW * nchunk * CHUNK - e
    if pad:
        row = jnp.pad(row, (0, pad))
        col = jnp.pad(col, (0, pad))
        edge_weight = jnp.pad(edge_weight, (0, pad))
    row2d = row.reshape(NW * nchunk, CHUNK)
    col2d = col.reshape(NW * nchunk, CHUNK)
    ew2d = edge_weight.reshape(NW * nchunk, CHUNK)
    zrows = jnp.zeros((RPT, DIM), jnp.float32)

    sweep = _make_sweep(nchunk)
    z = _dense_pre(x)
    p = sweep(z, col2d, row2d, ew2d, zrows)
    z = _dense_stage(_mid_body, p)
    p = sweep(z, col2d, row2d, ew2d, zrows)
    return _dense_stage(_post_body, p)


# speedup vs baseline: 1.3894x; 1.1164x over previous
import functools

import jax
import jax.numpy as jnp
from jax import lax
from jax.experimental import pallas as pl
from jax.experimental.pallas import tpu as pltpu
from jax.experimental.pallas import tpu_sc as plsc

N = 10000
DIM = 128
NC = 2
NS = 16
NW = NC * NS
L = 16
CHUNK = 128
SB = 8
RPT = 624
TAIL = N - NS * RPT


def _sweep_body(z, colr, rowr, ewr, zrows, out,
                colv0, rowv0, ewv0, rows0, colv1, rowv1, ewv1, rows1,
                acc, gs0, gs1, is0, is1, *, nchunk):
    colv = (colv0, colv1)
    rowv = (rowv0, rowv1)
    ewv = (ewv0, ewv1)
    rows = (rows0, rows1)
    gs = (gs0, gs1)
    isem = (is0, is1)
    c = lax.axis_index("c")
    s = lax.axis_index("s")
    wid = c * NS + s
    tbase = wid * nchunk * CHUNK

    def idx_start(i, p):
        base = pl.multiple_of(tbase + i * CHUNK, 8)
        pltpu.async_copy(colr.at[pl.ds(base, CHUNK)], colv[p], isem[p])
        pltpu.async_copy(rowr.at[pl.ds(base, CHUNK)], rowv[p], isem[p])
        pltpu.async_copy(ewr.at[pl.ds(base, CHUNK)], ewv[p], isem[p])

    def idx_wait(p):
        pltpu.make_async_copy(colr.at[pl.ds(0, CHUNK)], colv[p],
                              isem[p]).wait()
        pltpu.make_async_copy(rowr.at[pl.ds(0, CHUNK)], rowv[p],
                              isem[p]).wait()
        pltpu.make_async_copy(ewr.at[pl.ds(0, CHUNK)], ewv[p],
                              isem[p]).wait()

    idx_start(0, 0)
    idx_wait(0)
    pltpu.async_copy(z.at[colv[0]], rows[0], gs[0])
    idx_start(1, 1)

    zbase = pl.multiple_of(s * RPT, 8)
    pltpu.sync_copy(zrows.at[pl.ds(0, RPT)], acc.at[pl.ds(zbase, RPT)])

    @pl.when(s == NS - 1)
    def _():
        pltpu.sync_copy(zrows.at[pl.ds(0, TAIL)], acc.at[pl.ds(NS * RPT, TAIL)])

    plsc.subcore_barrier()

    def pair(gp, carry):
        for p in range(2):
            i = gp * 2 + p
            q = 1 - p
            pltpu.make_async_copy(z.at[colv[p]], rows[p], gs[p]).wait()

            @pl.when(i + 1 < nchunk)
            def _(p=p, q=q):
                idx_wait(q)
                pltpu.async_copy(z.at[colv[q]], rows[q], gs[q])

            def grp(g2, carry2, _p=p):
                wvec = ewv[_p][pl.ds(g2 * L, L)]
                for t in range(L):
                    wv = jnp.full((L,), wvec[t], jnp.float32)
                    e = g2 * L + t
                    for k in range(DIM // L):
                        rows[_p][e, pl.ds(k * L, L)] = (
                            rows[_p][e, pl.ds(k * L, L)] * wv)
                return carry2

            lax.fori_loop(0, CHUNK // L, grp, 0)

            pltpu.sync_copy(rows[p], acc.at[rowv[p]], add=True)

            @pl.when(i + 2 < nchunk)
            def _(i=i, p=p):
                idx_start(i + 2, p)

        return carry

    lax.fori_loop(0, nchunk // 2, pair, 0)

    plsc.subcore_barrier()
    dbase = pl.multiple_of(s * RPT, 8)
    pltpu.sync_copy(acc.at[pl.ds(dbase, RPT)], out.at[c, pl.ds(dbase, RPT)])

    @pl.when(s == NS - 1)
    def _():
        pltpu.sync_copy(acc.at[pl.ds(NS * RPT, TAIL)],
                        out.at[c, pl.ds(NS * RPT, TAIL)])


def _make_sweep(nchunk):
    mesh = plsc.VectorSubcoreMesh(core_axis_name="c", subcore_axis_name="s",
                                  num_cores=NC, num_subcores=NS)
    return pl.kernel(
        functools.partial(_sweep_body, nchunk=nchunk),
        out_type=jax.ShapeDtypeStruct((NC, N, DIM), jnp.float32),
        mesh=mesh,
        scratch_types=[
            pltpu.VMEM((CHUNK,), jnp.int32),
            pltpu.VMEM((CHUNK,), jnp.int32),
            pltpu.VMEM((CHUNK,), jnp.float32),
            pltpu.VMEM((CHUNK, DIM), jnp.float32),
            pltpu.VMEM((CHUNK,), jnp.int32),
            pltpu.VMEM((CHUNK,), jnp.int32),
            pltpu.VMEM((CHUNK,), jnp.float32),
            pltpu.VMEM((CHUNK, DIM), jnp.float32),
            pltpu.VMEM_SHARED((N, DIM), jnp.float32),
        ] + [pltpu.SemaphoreType.DMA] * 4,
    )


def _pre_body(x_ref, z_ref):
    x = x_ref[...]
    head = x[:, 0:1]
    tail = x[:, 1:]
    xk = tail / head
    n2 = jnp.clip(jnp.sum(xk * xk, axis=1, keepdims=True), 0.0, 0.9)
    lamb = 1.0 / jnp.sqrt(1.0 - n2)
    z_ref[...] = jnp.concatenate([lamb, lamb * xk], axis=1)


def _combine(p):
    a = p[0] + p[1]
    a0 = a[:, 0:1]
    inv = jnp.where(a0 != 0.0, 1.0 / a0, 0.0)
    km = a[:, 1:] * inv
    n2 = jnp.clip(jnp.sum(km * km, axis=1, keepdims=True), 0.0, 0.9)
    lamb = 1.0 / jnp.sqrt(1.0 - n2)
    pm = km * (lamb / (lamb + 1.0))
    alpha = 1.6732632423543772
    scale = 1.0507009873554805
    sp = scale * jnp.where(pm > 0, pm, alpha * (jnp.exp(pm) - 1.0))
    n2s = jnp.sum(sp * sp, axis=1, keepdims=True)
    denom = jnp.maximum(1.0 - n2s, 1e-6)
    xr = 2.0 * sp / denom
    headn = jnp.sqrt(1.0 + jnp.sum(xr * xr, axis=1, keepdims=True))
    return xr, headn


def _mid_body(p_ref, z_ref):
    xr, headn = _combine(p_ref[...])
    xk = xr / headn
    n2 = jnp.clip(jnp.sum(xk * xk, axis=1, keepdims=True), 0.0, 0.9)
    lamb = 1.0 / jnp.sqrt(1.0 - n2)
    z_ref[...] = jnp.concatenate([lamb, lamb * xk], axis=1)


def _post_body(p_ref, o_ref):
    xr, headn = _combine(p_ref[...])
    o_ref[...] = jnp.concatenate([headn, xr], axis=1)


_BLK = 1000


def _dense_pre(x):
    return pl.pallas_call(
        _pre_body,
        grid=(N // _BLK,),
        in_specs=[pl.BlockSpec((_BLK, DIM), lambda i: (i, 0))],
        out_specs=pl.BlockSpec((_BLK, DIM), lambda i: (i, 0)),
        out_shape=jax.ShapeDtypeStruct((N, DIM), jnp.float32),
    )(x)


def _dense_stage(body, p):
    return pl.pallas_call(
        body,
        grid=(N // _BLK,),
        in_specs=[pl.BlockSpec((NC, _BLK, DIM), lambda i: (0, i, 0))],
        out_specs=pl.BlockSpec((_BLK, DIM), lambda i: (i, 0)),
        out_shape=jax.ShapeDtypeStruct((N, DIM), jnp.float32),
    )(p)


def kernel(x, edge_index, edge_weight, msg_weight):
    del msg_weight
    row = edge_index[0]
    col = edge_index[1]
    e = edge_weight.shape[0]
    nchunk = -(-e // (NW * CHUNK))
    nchunk = -(-nchunk // 2) * 2
    pad = NW * nchunk * CHUNK - e
    if pad:
        row = jnp.pad(row, (0, pad))
        col = jnp.pad(col, (0, pad))
        edge_weight = jnp.pad(edge_weight, (0, pad))
    zrows = jnp.zeros((RPT, DIM), jnp.float32)

    sweep = _make_sweep(nchunk)
    z = _dense_pre(x)
    p = sweep(z, col, row, edge_weight, zrows)
    z = _dense_stage(_mid_body, p)
    p = sweep(z, col, row, edge_weight, zrows)
    return _dense_stage(_post_body, p)


# speedup vs baseline: 1.7147x; 1.2341x over previous
import functools

import jax
import jax.numpy as jnp
from jax import lax
from jax.experimental import pallas as pl
from jax.experimental.pallas import tpu as pltpu
from jax.experimental.pallas import tpu_sc as plsc

N = 10000
DIM = 128
NC = 2
NS = 16
NW = NC * NS
L = 16
CHUNK = 128
NSPLIT = 4
SUB = CHUNK // NSPLIT
RPT = 624
TAIL = N - NS * RPT


def _sweep_body(z, colr, rowr, ewr, zrows, out,
                cv0, cv1, cv2, cv3, rowv, ewv, rows,
                acc, g0, g1, g2, g3, isem, *, nchunk):
    colv = (cv0, cv1, cv2, cv3)
    gs = (g0, g1, g2, g3)
    c = lax.axis_index("c")
    s = lax.axis_index("s")
    wid = c * NS + s
    tbase = wid * nchunk * CHUNK

    zbase = pl.multiple_of(s * RPT, 8)
    pltpu.sync_copy(zrows.at[pl.ds(0, RPT)], acc.at[pl.ds(zbase, RPT)])

    @pl.when(s == NS - 1)
    def _():
        pltpu.sync_copy(zrows.at[pl.ds(0, TAIL)], acc.at[pl.ds(NS * RPT, TAIL)])

    plsc.subcore_barrier()

    def chunk_body(i, carry):
        base = pl.multiple_of(tbase + i * CHUNK, 8)
        for k in range(NSPLIT):
            pltpu.async_copy(colr.at[pl.ds(base + k * SUB, SUB)],
                             colv[k], isem)
        pltpu.async_copy(rowr.at[pl.ds(base, CHUNK)], rowv, isem)
        pltpu.async_copy(ewr.at[pl.ds(base, CHUNK)], ewv, isem)
        for k in range(NSPLIT):
            pltpu.make_async_copy(colr.at[pl.ds(0, SUB)], colv[k],
                                  isem).wait()
        pltpu.make_async_copy(rowr.at[pl.ds(0, CHUNK)], rowv, isem).wait()
        pltpu.make_async_copy(ewr.at[pl.ds(0, CHUNK)], ewv, isem).wait()

        for k in range(NSPLIT):
            pltpu.async_copy(z.at[colv[k]], rows.at[pl.ds(k * SUB, SUB)],
                             gs[k])
        for k in range(NSPLIT):
            pltpu.make_async_copy(z.at[colv[k]],
                                  rows.at[pl.ds(k * SUB, SUB)], gs[k]).wait()

        def grp(g2_, carry2):
            wvec = ewv[pl.ds(g2_ * L, L)]
            for t in range(L):
                wv = jnp.full((L,), wvec[t], jnp.float32)
                e = g2_ * L + t
                for j in range(DIM // L):
                    rows[e, pl.ds(j * L, L)] = rows[e, pl.ds(j * L, L)] * wv
            return carry2

        lax.fori_loop(0, CHUNK // L, grp, 0)

        pltpu.sync_copy(rows, acc.at[rowv], add=True)
        return carry

    lax.fori_loop(0, nchunk, chunk_body, 0)

    plsc.subcore_barrier()
    dbase = pl.multiple_of(s * RPT, 8)
    pltpu.sync_copy(acc.at[pl.ds(dbase, RPT)], out.at[c, pl.ds(dbase, RPT)])

    @pl.when(s == NS - 1)
    def _():
        pltpu.sync_copy(acc.at[pl.ds(NS * RPT, TAIL)],
                        out.at[c, pl.ds(NS * RPT, TAIL)])


def _make_sweep(nchunk):
    mesh = plsc.VectorSubcoreMesh(core_axis_name="c", subcore_axis_name="s",
                                  num_cores=NC, num_subcores=NS)
    return pl.kernel(
        functools.partial(_sweep_body, nchunk=nchunk),
        out_type=jax.ShapeDtypeStruct((NC, N, DIM), jnp.float32),
        mesh=mesh,
        scratch_types=[pltpu.VMEM((SUB,), jnp.int32)] * NSPLIT + [
            pltpu.VMEM((CHUNK,), jnp.int32),
            pltpu.VMEM((CHUNK,), jnp.float32),
            pltpu.VMEM((CHUNK, DIM), jnp.float32),
            pltpu.VMEM_SHARED((N, DIM), jnp.float32),
        ] + [pltpu.SemaphoreType.DMA] * (NSPLIT + 1),
    )


def _pre_body(x_ref, z_ref):
    x = x_ref[...]
    head = x[:, 0:1]
    tail = x[:, 1:]
    xk = tail / head
    n2 = jnp.clip(jnp.sum(xk * xk, axis=1, keepdims=True), 0.0, 0.9)
    lamb = 1.0 / jnp.sqrt(1.0 - n2)
    z_ref[...] = jnp.concatenate([lamb, lamb * xk], axis=1)


def _combine(p):
    a = p[0] + p[1]
    a0 = a[:, 0:1]
    inv = jnp.where(a0 != 0.0, 1.0 / a0, 0.0)
    km = a[:, 1:] * inv
    n2 = jnp.clip(jnp.sum(km * km, axis=1, keepdims=True), 0.0, 0.9)
    lamb = 1.0 / jnp.sqrt(1.0 - n2)
    pm = km * (lamb / (lamb + 1.0))
    alpha = 1.6732632423543772
    scale = 1.0507009873554805
    sp = scale * jnp.where(pm > 0, pm, alpha * (jnp.exp(pm) - 1.0))
    n2s = jnp.sum(sp * sp, axis=1, keepdims=True)
    denom = jnp.maximum(1.0 - n2s, 1e-6)
    xr = 2.0 * sp / denom
    headn = jnp.sqrt(1.0 + jnp.sum(xr * xr, axis=1, keepdims=True))
    return xr, headn


def _mid_body(p_ref, z_ref):
    xr, headn = _combine(p_ref[...])
    xk = xr / headn
    n2 = jnp.clip(jnp.sum(xk * xk, axis=1, keepdims=True), 0.0, 0.9)
    lamb = 1.0 / jnp.sqrt(1.0 - n2)
    z_ref[...] = jnp.concatenate([lamb, lamb * xk], axis=1)


def _post_body(p_ref, o_ref):
    xr, headn = _combine(p_ref[...])
    o_ref[...] = jnp.concatenate([headn, xr], axis=1)


_BLK = 1000


def _dense_pre(x):
    return pl.pallas_call(
        _pre_body,
        grid=(N // _BLK,),
        in_specs=[pl.BlockSpec((_BLK, DIM), lambda i: (i, 0))],
        out_specs=pl.BlockSpec((_BLK, DIM), lambda i: (i, 0)),
        out_shape=jax.ShapeDtypeStruct((N, DIM), jnp.float32),
    )(x)


def _dense_stage(body, p):
    return pl.pallas_call(
        body,
        grid=(N // _BLK,),
        in_specs=[pl.BlockSpec((NC, _BLK, DIM), lambda i: (0, i, 0))],
        out_specs=pl.BlockSpec((_BLK, DIM), lambda i: (i, 0)),
        out_shape=jax.ShapeDtypeStruct((N, DIM), jnp.float32),
    )(p)


def kernel(x, edge_index, edge_weight, msg_weight):
    del msg_weight
    row = edge_index[0]
    col = edge_index[1]
    e = edge_weight.shape[0]
    nchunk = -(-e // (NW * CHUNK))
    pad = NW * nchunk * CHUNK - e
    if pad:
        row = jnp.pad(row, (0, pad))
        col = jnp.pad(col, (0, pad))
        edge_weight = jnp.pad(edge_weight, (0, pad))
    zrows = jnp.zeros((RPT, DIM), jnp.float32)

    sweep = _make_sweep(nchunk)
    z = _dense_pre(x)
    p = sweep(z, col, row, edge_weight, zrows)
    z = _dense_stage(_mid_body, p)
    p = sweep(z, col, row, edge_weight, zrows)
    return _dense_stage(_post_body, p)


# speedup vs baseline: 1.7152x; 1.0003x over previous
import functools

import jax
import jax.numpy as jnp
from jax import lax
from jax.experimental import pallas as pl
from jax.experimental.pallas import tpu as pltpu
from jax.experimental.pallas import tpu_sc as plsc

N = 10000
DIM = 128
NC = 2
NS = 16
NW = NC * NS
L = 16
CHUNK = 128
NSPLIT = 8
SUB = CHUNK // NSPLIT
RPT = 624
TAIL = N - NS * RPT


def _sweep_body(z, colr, rowr, ewr, zrows, out,
                cv0, cv1, cv2, cv3, cv4, cv5, cv6, cv7, rowv, ewv, rows,
                acc, g0, g1, g2, g3, g4, g5, g6, g7, isem, *, nchunk):
    colv = (cv0, cv1, cv2, cv3, cv4, cv5, cv6, cv7)
    gs = (g0, g1, g2, g3, g4, g5, g6, g7)
    c = lax.axis_index("c")
    s = lax.axis_index("s")
    wid = c * NS + s
    tbase = wid * nchunk * CHUNK

    zbase = pl.multiple_of(s * RPT, 8)
    pltpu.sync_copy(zrows.at[pl.ds(0, RPT)], acc.at[pl.ds(zbase, RPT)])

    @pl.when(s == NS - 1)
    def _():
        pltpu.sync_copy(zrows.at[pl.ds(0, TAIL)], acc.at[pl.ds(NS * RPT, TAIL)])

    plsc.subcore_barrier()

    def chunk_body(i, carry):
        base = pl.multiple_of(tbase + i * CHUNK, 8)
        for k in range(NSPLIT):
            pltpu.async_copy(colr.at[pl.ds(base + k * SUB, SUB)],
                             colv[k], isem)
        pltpu.async_copy(rowr.at[pl.ds(base, CHUNK)], rowv, isem)
        pltpu.async_copy(ewr.at[pl.ds(base, CHUNK)], ewv, isem)
        for k in range(NSPLIT):
            pltpu.make_async_copy(colr.at[pl.ds(0, SUB)], colv[k],
                                  isem).wait()
        pltpu.make_async_copy(rowr.at[pl.ds(0, CHUNK)], rowv, isem).wait()
        pltpu.make_async_copy(ewr.at[pl.ds(0, CHUNK)], ewv, isem).wait()

        for k in range(NSPLIT):
            pltpu.async_copy(z.at[colv[k]], rows.at[pl.ds(k * SUB, SUB)],
                             gs[k])
        for k in range(NSPLIT):
            pltpu.make_async_copy(z.at[colv[k]],
                                  rows.at[pl.ds(k * SUB, SUB)], gs[k]).wait()

        def grp(g2_, carry2):
            wvec = ewv[pl.ds(g2_ * L, L)]
            for t in range(L):
                wv = jnp.full((L,), wvec[t], jnp.float32)
                e = g2_ * L + t
                for j in range(DIM // L):
                    rows[e, pl.ds(j * L, L)] = rows[e, pl.ds(j * L, L)] * wv
            return carry2

        lax.fori_loop(0, CHUNK // L, grp, 0)

        pltpu.sync_copy(rows, acc.at[rowv], add=True)
        return carry

    lax.fori_loop(0, nchunk, chunk_body, 0)

    plsc.subcore_barrier()
    dbase = pl.multiple_of(s * RPT, 8)
    pltpu.sync_copy(acc.at[pl.ds(dbase, RPT)], out.at[c, pl.ds(dbase, RPT)])

    @pl.when(s == NS - 1)
    def _():
        pltpu.sync_copy(acc.at[pl.ds(NS * RPT, TAIL)],
                        out.at[c, pl.ds(NS * RPT, TAIL)])


def _make_sweep(nchunk):
    mesh = plsc.VectorSubcoreMesh(core_axis_name="c", subcore_axis_name="s",
                                  num_cores=NC, num_subcores=NS)
    return pl.kernel(
        functools.partial(_sweep_body, nchunk=nchunk),
        out_type=jax.ShapeDtypeStruct((NC, N, DIM), jnp.float32),
        mesh=mesh,
        scratch_types=[pltpu.VMEM((SUB,), jnp.int32)] * NSPLIT + [
            pltpu.VMEM((CHUNK,), jnp.int32),
            pltpu.VMEM((CHUNK,), jnp.float32),
            pltpu.VMEM((CHUNK, DIM), jnp.float32),
            pltpu.VMEM_SHARED((N, DIM), jnp.float32),
        ] + [pltpu.SemaphoreType.DMA] * (NSPLIT + 1),
    )


def _pre_body(x_ref, z_ref):
    x = x_ref[...]
    head = x[:, 0:1]
    tail = x[:, 1:]
    xk = tail / head
    n2 = jnp.clip(jnp.sum(xk * xk, axis=1, keepdims=True), 0.0, 0.9)
    lamb = 1.0 / jnp.sqrt(1.0 - n2)
    z_ref[...] = jnp.concatenate([lamb, lamb * xk], axis=1)


def _combine(p):
    a = p[0] + p[1]
    a0 = a[:, 0:1]
    inv = jnp.where(a0 != 0.0, 1.0 / a0, 0.0)
    km = a[:, 1:] * inv
    n2 = jnp.clip(jnp.sum(km * km, axis=1, keepdims=True), 0.0, 0.9)
    lamb = 1.0 / jnp.sqrt(1.0 - n2)
    pm = km * (lamb / (lamb + 1.0))
    alpha = 1.6732632423543772
    scale = 1.0507009873554805
    sp = scale * jnp.where(pm > 0, pm, alpha * (jnp.exp(pm) - 1.0))
    n2s = jnp.sum(sp * sp, axis=1, keepdims=True)
    denom = jnp.maximum(1.0 - n2s, 1e-6)
    xr = 2.0 * sp / denom
    headn = jnp.sqrt(1.0 + jnp.sum(xr * xr, axis=1, keepdims=True))
    return xr, headn


def _mid_body(p_ref, z_ref):
    xr, headn = _combine(p_ref[...])
    xk = xr / headn
    n2 = jnp.clip(jnp.sum(xk * xk, axis=1, keepdims=True), 0.0, 0.9)
    lamb = 1.0 / jnp.sqrt(1.0 - n2)
    z_ref[...] = jnp.concatenate([lamb, lamb * xk], axis=1)


def _post_body(p_ref, o_ref):
    xr, headn = _combine(p_ref[...])
    o_ref[...] = jnp.concatenate([headn, xr], axis=1)


_BLK = 1000


def _dense_pre(x):
    return pl.pallas_call(
        _pre_body,
        grid=(N // _BLK,),
        in_specs=[pl.BlockSpec((_BLK, DIM), lambda i: (i, 0))],
        out_specs=pl.BlockSpec((_BLK, DIM), lambda i: (i, 0)),
        out_shape=jax.ShapeDtypeStruct((N, DIM), jnp.float32),
    )(x)


def _dense_stage(body, p):
    return pl.pallas_call(
        body,
        grid=(N // _BLK,),
        in_specs=[pl.BlockSpec((NC, _BLK, DIM), lambda i: (0, i, 0))],
        out_specs=pl.BlockSpec((_BLK, DIM), lambda i: (i, 0)),
        out_shape=jax.ShapeDtypeStruct((N, DIM), jnp.float32),
    )(p)


def kernel(x, edge_index, edge_weight, msg_weight):
    del msg_weight
    row = edge_index[0]
    col = edge_index[1]
    e = edge_weight.shape[0]
    nchunk = -(-e // (NW * CHUNK))
    pad = NW * nchunk * CHUNK - e
    if pad:
        row = jnp.pad(row, (0, pad))
        col = jnp.pad(col, (0, pad))
        edge_weight = jnp.pad(edge_weight, (0, pad))
    zrows = jnp.zeros((RPT, DIM), jnp.float32)

    sweep = _make_sweep(nchunk)
    z = _dense_pre(x)
    p = sweep(z, col, row, edge_weight, zrows)
    z = _dense_stage(_mid_body, p)
    p = sweep(z, col, row, edge_weight, zrows)
    return _dense_stage(_post_body, p)
